# baseline (speedup 1.0000x reference)
import functools

import jax
import jax.numpy as jnp
from jax import lax
from jax.experimental import pallas as pl
from jax.experimental.pallas import tpu as pltpu
from jax.experimental.pallas import tpu_sc as plsc

NC = 2
NS = 16
NW = NC * NS


@functools.lru_cache(maxsize=None)
def _make_gather(V, D, B, L):
    assert B % NW == 0
    nb = B // NW
    assert nb >= 4 and nb % 2 == 0
    mesh = plsc.VectorSubcoreMesh(core_axis_name="c", subcore_axis_name="s")

    @functools.partial(
        pl.kernel,
        out_type=jax.ShapeDtypeStruct((B, L, D), jnp.float32),
        mesh=mesh,
        compiler_params=pltpu.CompilerParams(use_tc_tiling_on_sc=False),
        scratch_types=[
            pltpu.VMEM((nb, L), jnp.int32),
            pltpu.VMEM((2, L, D), jnp.float32),
            pltpu.SemaphoreType.DMA,
            pltpu.SemaphoreType.DMA,
            pltpu.SemaphoreType.DMA,
        ],
    )
    def gather_kernel(table_hbm, idx_hbm, out_hbm, idx_v, rows_v, g0sem, g1sem, ssem):
        wid = lax.axis_index("s") * NC + lax.axis_index("c")
        base = wid * nb
        pltpu.sync_copy(idx_hbm.at[pl.ds(base, nb)], idx_v)
        gsems = (g0sem, g1sem)

        def fire_gather(g, parity):
            pltpu.async_copy(
                table_hbm.at[idx_v.at[g]], rows_v.at[parity], gsems[parity])

        def drain_gather(g, parity):
            pltpu.make_async_copy(
                table_hbm.at[idx_v.at[g]], rows_v.at[parity], gsems[parity]
            ).wait()

        def fire_store(g, parity):
            pltpu.async_copy(rows_v.at[parity], out_hbm.at[base + g], ssem)

        def wait_store(g, parity):
            pltpu.make_async_copy(
                rows_v.at[parity], out_hbm.at[base + g], ssem
            ).wait()

        def steady_step(g, parity):
            wait_store(g - 1, 1 - parity)
            fire_gather(g + 1, 1 - parity)
            drain_gather(g, parity)
            fire_store(g, parity)

        fire_gather(0, 0)
        fire_gather(1, 1)
        drain_gather(0, 0)
        fire_store(0, 0)

        @pl.loop(0, (nb - 2) // 2)
        def _(p):
            g = 2 * p + 1
            steady_step(g, 1)
            steady_step(g + 1, 0)

        last = nb - 1
        wait_store(last - 1, (last - 1) % 2)
        drain_gather(last, last % 2)
        fire_store(last, last % 2)
        wait_store(last, last % 2)

    return gather_kernel


def kernel(sequence, src_word_table):
    batch, seq_len = sequence.shape
    vocab, emsize = src_word_table.shape
    return _make_gather(vocab, emsize, batch, seq_len)(src_word_table, sequence)
